# baseline (speedup 1.0000x reference)
import functools

import jax
import jax.numpy as jnp
from jax import lax
from jax.experimental import pallas as pl
from jax.experimental.pallas import tpu as pltpu
from jax.experimental.pallas import tpu_sc as plsc

N_NODES = 10000
NPAD = 10240
RB = 400
GRID = N_NODES // RB
NC, NS = 2, 16
NW = NC * NS
RPT = NPAD // NS
CH = 128
PCH = 80
PHASES = 2
TOTC = PHASES * PCH + 2

_MESH = plsc.VectorSubcoreMesh(core_axis_name="c", subcore_axis_name="s")


def _sc_deg(dstp, ewp):
    @functools.partial(
        pl.kernel,
        out_type=(jax.ShapeDtypeStruct((NPAD,), jnp.float32),
                  jax.ShapeDtypeStruct((NPAD,), jnp.float32)),
        mesh=_MESH,
        scratch_types=[
            pltpu.VMEM((PCH + 2, CH), jnp.int32),
            pltpu.VMEM((PCH + 2, CH), jnp.float32),
            pltpu.VMEM((CH,), jnp.float32),
            pltpu.VMEM_SHARED((NPAD,), jnp.float32),
        ],
        compiler_params=pltpu.CompilerParams(use_tc_tiling_on_sc=False),
    )
    def k(dst_hbm, ew_hbm, out0, out1, dst_big, ew_big, zv, acc):
        cid = lax.axis_index("c")
        sid = lax.axis_index("s")
        pltpu.sync_copy(dst_hbm.at[sid, pl.ds(cid * PCH, PCH + 2)], dst_big)
        pltpu.sync_copy(ew_hbm.at[sid, pl.ds(cid * PCH, PCH + 2)], ew_big)
        for c in range(CH // 16):
            zv[pl.ds(c * 16, 16)] = jnp.zeros((16,), jnp.float32)
        for i in range(RPT // CH):
            pltpu.sync_copy(zv, acc.at[pl.ds(sid * RPT + i * CH, CH)])
        plsc.subcore_barrier()

        def body(kk, carry):
            pltpu.sync_copy(ew_big.at[kk], acc.at[dst_big.at[kk]], add=True)
            return carry

        lax.fori_loop(0, PCH, body, 0)
        plsc.subcore_barrier()

        @pl.when(cid == 0)
        def _():
            pltpu.sync_copy(acc.at[pl.ds(sid * RPT, RPT)],
                            out0.at[pl.ds(sid * RPT, RPT)])

        @pl.when(cid == 1)
        def _():
            pltpu.sync_copy(acc.at[pl.ds(sid * RPT, RPT)],
                            out1.at[pl.ds(sid * RPT, RPT)])

    return k(dstp, ewp)


def _sc_msg(srcp, dstp, ewp, gstk, h):
    @functools.partial(
        pl.kernel,
        out_type=(jax.ShapeDtypeStruct((NPAD, h), jnp.float32),
                  jax.ShapeDtypeStruct((NPAD, h), jnp.float32)),
        mesh=_MESH,
        scratch_types=[
            pltpu.VMEM((PCH + 2, CH), jnp.int32),
            pltpu.VMEM((PCH + 2, CH), jnp.int32),
            pltpu.VMEM((PCH + 2, CH), jnp.float32),
            pltpu.VMEM((CH, h), jnp.float32),
            pltpu.VMEM((CH, h), jnp.float32),
            pltpu.VMEM((CH, h), jnp.float32),
            pltpu.VMEM((CH, h), jnp.float32),
            pltpu.VMEM_SHARED((NPAD, h), jnp.float32),
            pltpu.SemaphoreType.DMA,
            pltpu.SemaphoreType.DMA,
            pltpu.SemaphoreType.DMA,
            pltpu.SemaphoreType.DMA,
        ],
        compiler_params=pltpu.CompilerParams(use_tc_tiling_on_sc=False),
    )
    def k(src_hbm, dst_hbm, ew_hbm, g_hbm, out0, out1,
          src_big, dst_big, ew_big, rows0, rows1, msg0, msg1, acc,
          sg0, sg1, ss0, ss1):
        cid = lax.axis_index("c")
        sid = lax.axis_index("s")

        rows = (rows0, rows1)
        msgs = (msg0, msg1)
        sgs = (sg0, sg1)
        sss = (ss0, ss1)

        def stage(p):
            pltpu.sync_copy(src_hbm.at[sid, pl.ds(p * PCH, PCH + 2)], src_big)
            pltpu.sync_copy(dst_hbm.at[sid, pl.ds(p * PCH, PCH + 2)], dst_big)
            pltpu.sync_copy(ew_hbm.at[sid, pl.ds(p * PCH, PCH + 2)], ew_big)
            off = jnp.zeros((16,), jnp.int32) + cid * NPAD

            def obody(r, carry):
                for c in range(CH // 16):
                    sl = pl.ds(c * 16, 16)
                    src_big[r, sl] = src_big[r, sl] + off
                return carry

            lax.fori_loop(0, PCH + 2, obody, 0)

        stage(0)
        pltpu.async_copy(g_hbm.at[src_big.at[0]], rows0, sg0)
        pltpu.async_copy(g_hbm.at[src_big.at[1]], rows1, sg1)

        def zbody(j, carry):
            for c in range(h // 16):
                msg0[j, pl.ds(c * 16, 16)] = jnp.zeros((16,), jnp.float32)
            return carry

        lax.fori_loop(0, CH, zbody, 0)
        for i in range(RPT // CH):
            pltpu.sync_copy(msg0, acc.at[pl.ds(sid * RPT + i * CH, CH), :])
        plsc.subcore_barrier()

        for p in range(PHASES):
            if p > 0:
                stage(p)
                pltpu.async_copy(g_hbm.at[src_big.at[0]], rows0, sg0)
                pltpu.async_copy(g_hbm.at[src_big.at[1]], rows1, sg1)

            def pair(kk, carry):
                for b in range(2):
                    kchunk = 2 * kk + b
                    pltpu.make_async_copy(g_hbm.at[src_big.at[kchunk]],
                                          rows[b], sgs[b]).wait()

                    @pl.when(kk >= 1)
                    def _():
                        pltpu.make_async_copy(msgs[b],
                                              acc.at[dst_big.at[kchunk]],
                                              sss[b]).wait()

                    def grp(gg, c2, b=b, kchunk=kchunk):
                        ew_g = ew_big[kchunk, pl.ds(gg * 16, 16)]
                        jb = gg * 16
                        for j2 in range(16):
                            w = lax.gather(
                                ew_g, jnp.full((16, 1), j2, jnp.int32),
                                lax.GatherDimensionNumbers(
                                    offset_dims=(), collapsed_slice_dims=(0,),
                                    start_index_map=(0,)),
                                (1,),
                                mode=lax.GatherScatterMode.PROMISE_IN_BOUNDS)
                            for c in range(h // 16):
                                sl = pl.ds(c * 16, 16)
                                msgs[b][jb + j2, sl] = (
                                    rows[b][jb + j2, sl] * w)
                        return c2

                    lax.fori_loop(0, CH // 16, grp, 0)
                    pltpu.async_copy(g_hbm.at[src_big.at[kchunk + 2]],
                                     rows[b], sgs[b])
                    pltpu.async_copy(msgs[b], acc.at[dst_big.at[kchunk]],
                                     sss[b], add=True)
                return carry

            lax.fori_loop(0, PCH // 2, pair, 0)

            for b in range(2):
                pltpu.make_async_copy(msgs[b],
                                      acc.at[dst_big.at[PCH - 2 + b]],
                                      sss[b]).wait()
                pltpu.make_async_copy(g_hbm.at[src_big.at[PCH + b]],
                                      rows[b], sgs[b]).wait()

        plsc.subcore_barrier()

        @pl.when(cid == 0)
        def _():
            pltpu.sync_copy(acc.at[pl.ds(sid * RPT, RPT), :],
                            out0.at[pl.ds(sid * RPT, RPT), :])

        @pl.when(cid == 1)
        def _():
            pltpu.sync_copy(acc.at[pl.ds(sid * RPT, RPT), :],
                            out1.at[pl.ds(sid * RPT, RPT), :])

    return k(srcp, dstp, ewp, gstk)


def _g_body(x_ref, w_ref, d0_ref, d1_ref, g_ref):
    dis = jax.lax.rsqrt(1.0 + d0_ref[...] + d1_ref[...])
    gfull = dis * jnp.dot(x_ref[...], w_ref[...],
                          preferred_element_type=jnp.float32)
    hh = gfull.shape[1] // 2
    g_ref[0] = gfull[:, :hh]
    g_ref[1] = gfull[:, hh:]


def _tc_g(x, W, d0, d1):
    h = W.shape[1]
    return pl.pallas_call(
        _g_body,
        grid=(GRID,),
        in_specs=[
            pl.BlockSpec((RB, x.shape[1]), lambda i: (i, 0)),
            pl.BlockSpec((W.shape[0], h), lambda i: (0, 0)),
            pl.BlockSpec((RB, 1), lambda i: (i, 0)),
            pl.BlockSpec((RB, 1), lambda i: (i, 0)),
        ],
        out_specs=pl.BlockSpec((2, RB, h // 2), lambda i: (0, i, 0)),
        out_shape=jax.ShapeDtypeStruct((2, NPAD, h // 2), jnp.float32),
    )(x, W, d0, d1)


def _mid_body(a0_ref, a1_ref, g_ref, d0_ref, d1_ref, w_ref, b_ref, g2_ref):
    dis = jax.lax.rsqrt(1.0 + d0_ref[...] + d1_ref[...])
    a = jnp.concatenate([a0_ref[...], a1_ref[...]], axis=1)
    g = jnp.concatenate([g_ref[0], g_ref[1]], axis=1)
    z = jax.nn.relu(dis * (a + g) + b_ref[...])
    h2 = dis * jnp.dot(z, w_ref[...], preferred_element_type=jnp.float32)
    hh = h2.shape[1] // 2
    g2_ref[0] = h2[:, :hh]
    g2_ref[1] = h2[:, hh:]


def _tc_mid(a0, a1, g, d0, d1, W, b):
    h0 = 2 * g.shape[2]
    h1 = W.shape[1]
    return pl.pallas_call(
        _mid_body,
        grid=(GRID,),
        in_specs=[
            pl.BlockSpec((RB, h0 // 2), lambda i: (i, 0)),
            pl.BlockSpec((RB, h0 // 2), lambda i: (i, 0)),
            pl.BlockSpec((2, RB, h0 // 2), lambda i: (0, i, 0)),
            pl.BlockSpec((RB, 1), lambda i: (i, 0)),
            pl.BlockSpec((RB, 1), lambda i: (i, 0)),
            pl.BlockSpec((h0, h1), lambda i: (0, 0)),
            pl.BlockSpec((1, h0), lambda i: (0, 0)),
        ],
        out_specs=pl.BlockSpec((2, RB, h1 // 2), lambda i: (0, i, 0)),
        out_shape=jax.ShapeDtypeStruct((2, NPAD, h1 // 2), jnp.float32),
    )(a0, a1, g, d0, d1, W, b)


def _fin_body(a0_ref, a1_ref, g_ref, d0_ref, d1_ref, b_ref, o_ref):
    dis = jax.lax.rsqrt(1.0 + d0_ref[...] + d1_ref[...])
    a = jnp.concatenate([a0_ref[...], a1_ref[...]], axis=1)
    g = jnp.concatenate([g_ref[0], g_ref[1]], axis=1)
    t = dis * (a + g) + b_ref[...]
    t = t - jnp.max(t, axis=1, keepdims=True)
    e = jnp.exp(t)
    o_ref[...] = e / jnp.sum(e, axis=1, keepdims=True)


def _tc_fin(a0, a1, g, d0, d1, b):
    h = 2 * g.shape[2]
    return pl.pallas_call(
        _fin_body,
        grid=(GRID,),
        in_specs=[
            pl.BlockSpec((RB, h // 2), lambda i: (i, 0)),
            pl.BlockSpec((RB, h // 2), lambda i: (i, 0)),
            pl.BlockSpec((2, RB, h // 2), lambda i: (0, i, 0)),
            pl.BlockSpec((RB, 1), lambda i: (i, 0)),
            pl.BlockSpec((RB, 1), lambda i: (i, 0)),
            pl.BlockSpec((1, h), lambda i: (0, 0)),
        ],
        out_specs=pl.BlockSpec((RB, h), lambda i: (i, 0)),
        out_shape=jax.ShapeDtypeStruct((N_NODES, h), jnp.float32),
    )(a0, a1, g, d0, d1, b)


def kernel(x, edge_index, edge_attr, W1, b1, W2, b2):
    src = edge_index[0].astype(jnp.int32)
    dst = edge_index[1].astype(jnp.int32)
    ew = edge_attr
    e_tot = src.shape[0]

    ept = e_tot // NS
    pad = TOTC * CH - ept
    psrc = jnp.broadcast_to(jnp.arange(pad, dtype=jnp.int32) % N_NODES,
                            (NS, pad))
    pdst = jnp.broadcast_to(
        (jnp.arange(pad, dtype=jnp.int32) % (NPAD - N_NODES)) + N_NODES,
        (NS, pad))
    srcp = jnp.concatenate([src.reshape(NS, ept), psrc],
                           axis=1).reshape(NS, TOTC, CH)
    dstp = jnp.concatenate([dst.reshape(NS, ept), pdst],
                           axis=1).reshape(NS, TOTC, CH)
    ewp = jnp.concatenate([ew.reshape(NS, ept),
                           jnp.zeros((NS, pad), jnp.float32)],
                          axis=1).reshape(NS, TOTC, CH)

    deg0, deg1 = _sc_deg(dstp, ewp)
    d0 = deg0[:N_NODES].reshape(N_NODES, 1)
    d1 = deg1[:N_NODES].reshape(N_NODES, 1)

    g1 = _tc_g(x, W1, d0, d1)
    a1_lo, a1_hi = _sc_msg(srcp, dstp, ewp, g1.reshape(2 * NPAD, 64), 64)
    g2 = _tc_mid(a1_lo, a1_hi, g1, d0, d1, W2, b1.reshape(1, -1))
    a2_lo, a2_hi = _sc_msg(srcp, dstp, ewp, g2.reshape(2 * NPAD, 32), 32)
    return _tc_fin(a2_lo, a2_hi, g2, d0, d1, b2.reshape(1, -1))


# speedup vs baseline: 1.0909x; 1.0909x over previous
import functools

import jax
import jax.numpy as jnp
from jax import lax
from jax.experimental import pallas as pl
from jax.experimental.pallas import tpu as pltpu
from jax.experimental.pallas import tpu_sc as plsc

N_NODES = 10000
NPAD = 10240
RB = 400
GRID = N_NODES // RB
NC, NS = 2, 16
NW = NC * NS
RPT = NPAD // NS
CH = 128
PCH = 80
PHASES = 2
TOTC = PHASES * PCH + 2

_MESH = plsc.VectorSubcoreMesh(core_axis_name="c", subcore_axis_name="s")


def _sc_deg(dstp, ewp):
    @functools.partial(
        pl.kernel,
        out_type=(jax.ShapeDtypeStruct((NPAD,), jnp.float32),
                  jax.ShapeDtypeStruct((NPAD,), jnp.float32)),
        mesh=_MESH,
        scratch_types=[
            pltpu.VMEM((PCH + 2, CH), jnp.int32),
            pltpu.VMEM((PCH + 2, CH), jnp.float32),
            pltpu.VMEM((CH,), jnp.float32),
            pltpu.VMEM_SHARED((NPAD,), jnp.float32),
        ],
        compiler_params=pltpu.CompilerParams(use_tc_tiling_on_sc=False),
    )
    def k(dst_hbm, ew_hbm, out0, out1, dst_big, ew_big, zv, acc):
        cid = lax.axis_index("c")
        sid = lax.axis_index("s")
        pltpu.sync_copy(dst_hbm.at[sid, pl.ds(cid * PCH, PCH + 2)], dst_big)
        pltpu.sync_copy(ew_hbm.at[sid, pl.ds(cid * PCH, PCH + 2)], ew_big)
        for c in range(CH // 16):
            zv[pl.ds(c * 16, 16)] = jnp.zeros((16,), jnp.float32)
        for i in range(RPT // CH):
            pltpu.sync_copy(zv, acc.at[pl.ds(sid * RPT + i * CH, CH)])
        plsc.subcore_barrier()

        def body(kk, carry):
            pltpu.sync_copy(ew_big.at[kk], acc.at[dst_big.at[kk]], add=True)
            return carry

        lax.fori_loop(0, PCH, body, 0)
        plsc.subcore_barrier()

        @pl.when(cid == 0)
        def _():
            pltpu.sync_copy(acc.at[pl.ds(sid * RPT, RPT)],
                            out0.at[pl.ds(sid * RPT, RPT)])

        @pl.when(cid == 1)
        def _():
            pltpu.sync_copy(acc.at[pl.ds(sid * RPT, RPT)],
                            out1.at[pl.ds(sid * RPT, RPT)])

    return k(dstp, ewp)


def _sc_msg(srcp, dstp, ewp, gstk, h, colsplit):
    phases = PHASES if colsplit else 1

    @functools.partial(
        pl.kernel,
        out_type=(jax.ShapeDtypeStruct((NPAD, h), jnp.float32),
                  jax.ShapeDtypeStruct((NPAD, h), jnp.float32)),
        mesh=_MESH,
        scratch_types=[
            pltpu.VMEM((PCH + 2, CH), jnp.int32),
            pltpu.VMEM((PCH + 2, CH), jnp.int32),
            pltpu.VMEM((PCH + 2, CH), jnp.float32),
            pltpu.VMEM((CH, h), jnp.float32),
            pltpu.VMEM((CH, h), jnp.float32),
            pltpu.VMEM((CH, h), jnp.float32),
            pltpu.VMEM((CH, h), jnp.float32),
            pltpu.VMEM_SHARED((NPAD, h), jnp.float32),
            pltpu.SemaphoreType.DMA,
            pltpu.SemaphoreType.DMA,
            pltpu.SemaphoreType.DMA,
            pltpu.SemaphoreType.DMA,
        ],
        compiler_params=pltpu.CompilerParams(use_tc_tiling_on_sc=False),
    )
    def k(src_hbm, dst_hbm, ew_hbm, g_hbm, out0, out1,
          src_big, dst_big, ew_big, rows0, rows1, msg0, msg1, acc,
          sg0, sg1, ss0, ss1):
        cid = lax.axis_index("c")
        sid = lax.axis_index("s")

        rows = (rows0, rows1)
        msgs = (msg0, msg1)
        sgs = (sg0, sg1)
        sss = (ss0, ss1)

        def stage(p):
            w0 = (p if colsplit else cid) * PCH
            pltpu.sync_copy(src_hbm.at[sid, pl.ds(w0, PCH + 2)], src_big)
            pltpu.sync_copy(dst_hbm.at[sid, pl.ds(w0, PCH + 2)], dst_big)
            pltpu.sync_copy(ew_hbm.at[sid, pl.ds(w0, PCH + 2)], ew_big)
            if colsplit:
                off = jnp.zeros((16,), jnp.int32) + cid * NPAD

                def obody(r, carry):
                    for c in range(CH // 16):
                        sl = pl.ds(c * 16, 16)
                        src_big[r, sl] = src_big[r, sl] + off
                    return carry

                lax.fori_loop(0, PCH + 2, obody, 0)

        stage(0)
        pltpu.async_copy(g_hbm.at[src_big.at[0]], rows0, sg0)
        pltpu.async_copy(g_hbm.at[src_big.at[1]], rows1, sg1)

        def zbody(j, carry):
            for c in range(h // 16):
                msg0[j, pl.ds(c * 16, 16)] = jnp.zeros((16,), jnp.float32)
            return carry

        lax.fori_loop(0, CH, zbody, 0)
        for i in range(RPT // CH):
            pltpu.sync_copy(msg0, acc.at[pl.ds(sid * RPT + i * CH, CH), :])
        plsc.subcore_barrier()

        for p in range(phases):
            if p > 0:
                stage(p)
                pltpu.async_copy(g_hbm.at[src_big.at[0]], rows0, sg0)
                pltpu.async_copy(g_hbm.at[src_big.at[1]], rows1, sg1)

            def pair(kk, carry):
                for b in range(2):
                    kchunk = 2 * kk + b
                    pltpu.make_async_copy(g_hbm.at[src_big.at[kchunk]],
                                          rows[b], sgs[b]).wait()

                    @pl.when(kk >= 1)
                    def _():
                        pltpu.make_async_copy(msgs[b],
                                              acc.at[dst_big.at[kchunk]],
                                              sss[b]).wait()

                    def grp(gg, c2, b=b, kchunk=kchunk):
                        ew_g = ew_big[kchunk, pl.ds(gg * 16, 16)]
                        jb = gg * 16
                        for j2 in range(16):
                            w = lax.gather(
                                ew_g, jnp.full((16, 1), j2, jnp.int32),
                                lax.GatherDimensionNumbers(
                                    offset_dims=(), collapsed_slice_dims=(0,),
                                    start_index_map=(0,)),
                                (1,),
                                mode=lax.GatherScatterMode.PROMISE_IN_BOUNDS)
                            for c in range(h // 16):
                                sl = pl.ds(c * 16, 16)
                                msgs[b][jb + j2, sl] = (
                                    rows[b][jb + j2, sl] * w)
                        return c2

                    lax.fori_loop(0, CH // 16, grp, 0)
                    pltpu.async_copy(g_hbm.at[src_big.at[kchunk + 2]],
                                     rows[b], sgs[b])
                    pltpu.async_copy(msgs[b], acc.at[dst_big.at[kchunk]],
                                     sss[b], add=True)
                return carry

            lax.fori_loop(0, PCH // 2, pair, 0)

            for b in range(2):
                pltpu.make_async_copy(msgs[b],
                                      acc.at[dst_big.at[PCH - 2 + b]],
                                      sss[b]).wait()
                pltpu.make_async_copy(g_hbm.at[src_big.at[PCH + b]],
                                      rows[b], sgs[b]).wait()

        plsc.subcore_barrier()

        @pl.when(cid == 0)
        def _():
            pltpu.sync_copy(acc.at[pl.ds(sid * RPT, RPT), :],
                            out0.at[pl.ds(sid * RPT, RPT), :])

        @pl.when(cid == 1)
        def _():
            pltpu.sync_copy(acc.at[pl.ds(sid * RPT, RPT), :],
                            out1.at[pl.ds(sid * RPT, RPT), :])

    return k(srcp, dstp, ewp, gstk)


def _g_body(x_ref, w_ref, d0_ref, d1_ref, g_ref):
    dis = jax.lax.rsqrt(1.0 + d0_ref[...] + d1_ref[...])
    gfull = dis * jnp.dot(x_ref[...], w_ref[...],
                          preferred_element_type=jnp.float32)
    hh = gfull.shape[1] // 2
    g_ref[0] = gfull[:, :hh]
    g_ref[1] = gfull[:, hh:]


def _tc_g(x, W, d0, d1):
    h = W.shape[1]
    return pl.pallas_call(
        _g_body,
        grid=(GRID,),
        in_specs=[
            pl.BlockSpec((RB, x.shape[1]), lambda i: (i, 0)),
            pl.BlockSpec((W.shape[0], h), lambda i: (0, 0)),
            pl.BlockSpec((RB, 1), lambda i: (i, 0)),
            pl.BlockSpec((RB, 1), lambda i: (i, 0)),
        ],
        out_specs=pl.BlockSpec((2, RB, h // 2), lambda i: (0, i, 0)),
        out_shape=jax.ShapeDtypeStruct((2, NPAD, h // 2), jnp.float32),
    )(x, W, d0, d1)


def _mid_body(a0_ref, a1_ref, g_ref, d0_ref, d1_ref, w_ref, b_ref, g2_ref):
    dis = jax.lax.rsqrt(1.0 + d0_ref[...] + d1_ref[...])
    a = jnp.concatenate([a0_ref[...], a1_ref[...]], axis=1)
    g = jnp.concatenate([g_ref[0], g_ref[1]], axis=1)
    z = jax.nn.relu(dis * (a + g) + b_ref[...])
    g2_ref[...] = dis * jnp.dot(z, w_ref[...],
                                preferred_element_type=jnp.float32)


def _tc_mid(a0, a1, g, d0, d1, W, b):
    h0 = 2 * g.shape[2]
    h1 = W.shape[1]
    return pl.pallas_call(
        _mid_body,
        grid=(GRID,),
        in_specs=[
            pl.BlockSpec((RB, h0 // 2), lambda i: (i, 0)),
            pl.BlockSpec((RB, h0 // 2), lambda i: (i, 0)),
            pl.BlockSpec((2, RB, h0 // 2), lambda i: (0, i, 0)),
            pl.BlockSpec((RB, 1), lambda i: (i, 0)),
            pl.BlockSpec((RB, 1), lambda i: (i, 0)),
            pl.BlockSpec((h0, h1), lambda i: (0, 0)),
            pl.BlockSpec((1, h0), lambda i: (0, 0)),
        ],
        out_specs=pl.BlockSpec((RB, h1), lambda i: (i, 0)),
        out_shape=jax.ShapeDtypeStruct((NPAD, h1), jnp.float32),
    )(a0, a1, g, d0, d1, W, b)


def _fin_body(a0_ref, a1_ref, g_ref, d0_ref, d1_ref, b_ref, o_ref):
    dis = jax.lax.rsqrt(1.0 + d0_ref[...] + d1_ref[...])
    t = dis * (a0_ref[...] + a1_ref[...] + g_ref[...]) + b_ref[...]
    t = t - jnp.max(t, axis=1, keepdims=True)
    e = jnp.exp(t)
    o_ref[...] = e / jnp.sum(e, axis=1, keepdims=True)


def _tc_fin(a0, a1, g, d0, d1, b):
    h = g.shape[1]
    return pl.pallas_call(
        _fin_body,
        grid=(GRID,),
        in_specs=[
            pl.BlockSpec((RB, h), lambda i: (i, 0)),
            pl.BlockSpec((RB, h), lambda i: (i, 0)),
            pl.BlockSpec((RB, h), lambda i: (i, 0)),
            pl.BlockSpec((RB, 1), lambda i: (i, 0)),
            pl.BlockSpec((RB, 1), lambda i: (i, 0)),
            pl.BlockSpec((1, h), lambda i: (0, 0)),
        ],
        out_specs=pl.BlockSpec((RB, h), lambda i: (i, 0)),
        out_shape=jax.ShapeDtypeStruct((N_NODES, h), jnp.float32),
    )(a0, a1, g, d0, d1, b)


def kernel(x, edge_index, edge_attr, W1, b1, W2, b2):
    src = edge_index[0].astype(jnp.int32)
    dst = edge_index[1].astype(jnp.int32)
    ew = edge_attr
    e_tot = src.shape[0]

    ept = e_tot // NS
    pad = TOTC * CH - ept
    psrc = jnp.broadcast_to(jnp.arange(pad, dtype=jnp.int32) % N_NODES,
                            (NS, pad))
    pdst = jnp.broadcast_to(
        (jnp.arange(pad, dtype=jnp.int32) % (NPAD - N_NODES)) + N_NODES,
        (NS, pad))
    srcp = jnp.concatenate([src.reshape(NS, ept), psrc],
                           axis=1).reshape(NS, TOTC, CH)
    dstp = jnp.concatenate([dst.reshape(NS, ept), pdst],
                           axis=1).reshape(NS, TOTC, CH)
    ewp = jnp.concatenate([ew.reshape(NS, ept),
                           jnp.zeros((NS, pad), jnp.float32)],
                          axis=1).reshape(NS, TOTC, CH)

    deg0, deg1 = _sc_deg(dstp, ewp)
    d0 = deg0[:N_NODES].reshape(N_NODES, 1)
    d1 = deg1[:N_NODES].reshape(N_NODES, 1)

    g1 = _tc_g(x, W1, d0, d1)
    a1_lo, a1_hi = _sc_msg(srcp, dstp, ewp, g1.reshape(2 * NPAD, 64),
                           64, True)
    g2 = _tc_mid(a1_lo, a1_hi, g1, d0, d1, W2, b1.reshape(1, -1))
    a2_0, a2_1 = _sc_msg(srcp, dstp, ewp, g2, 64, False)
    return _tc_fin(a2_0, a2_1, g2, d0, d1, b2.reshape(1, -1))
